# baseline (speedup 1.0000x reference)
import functools

import jax
import jax.numpy as jnp
from jax.experimental import pallas as pl
from jax.experimental.pallas import tpu as pltpu

_VMEM_LIMIT_BYTES = 48 * 1024 * 1024


def _se_hwbc_kernel(x_ref, w1t_ref, b1_ref, w2t_ref, b2_ref, y_ref):
    x = x_ref[...]
    hw = x_ref.shape[0]
    pooled = jnp.sum(x, axis=0, dtype=jnp.float32) * (1.0 / hw)
    h1 = jnp.dot(pooled, w1t_ref[...], preferred_element_type=jnp.float32)
    h1 = jnp.maximum(h1 + b1_ref[...], 0.0)
    h2 = jnp.dot(h1, w2t_ref[...], preferred_element_type=jnp.float32)
    gate = jax.nn.sigmoid(h2 + b2_ref[...])
    y_ref[...] = gate.astype(y_ref.dtype)[None, :, :] * x


@functools.partial(jax.jit, static_argnames=("bt",))
def _spatial_se(x, w1t, b1, w2t, b2, *, bt):
    B, C, H, W = x.shape
    hidden = w1t.shape[1]
    HW = H * W

    xt = jnp.transpose(x, (2, 3, 0, 1)).reshape(HW, B, C)
    nb = pl.cdiv(B, bt)

    yt = pl.pallas_call(
        _se_hwbc_kernel,
        out_shape=jax.ShapeDtypeStruct((HW, B, C), x.dtype),
        grid=(nb,),
        in_specs=[
            pl.BlockSpec((HW, bt, C), lambda b: (0, b, 0)),
            pl.BlockSpec((C, hidden), lambda b: (0, 0)),
            pl.BlockSpec((1, hidden), lambda b: (0, 0)),
            pl.BlockSpec((hidden, C), lambda b: (0, 0)),
            pl.BlockSpec((1, C), lambda b: (0, 0)),
        ],
        out_specs=pl.BlockSpec((HW, bt, C), lambda b: (0, b, 0)),
        compiler_params=pltpu.CompilerParams(
            dimension_semantics=("parallel",),
            vmem_limit_bytes=_VMEM_LIMIT_BYTES),
    )(xt, w1t, b1, w2t, b2)

    return yt.reshape(H, W, B, C).transpose(2, 3, 0, 1)


def kernel(x, w1t, b1, w2t, b2):
    return _spatial_se(x, w1t, b1, w2t, b2, bt=8)


# speedup vs baseline: 1.2083x; 1.2083x over previous
import functools

import jax
import jax.numpy as jnp
from jax.experimental import pallas as pl
from jax.experimental.pallas import tpu as pltpu

_VMEM_LIMIT_BYTES = 58 * 1024 * 1024


def _se_hwbc_kernel(x_ref, w1t_ref, b1_ref, w2t_ref, b2_ref, y_ref):
    x = x_ref[...]
    hw = x_ref.shape[0]
    pooled = jnp.sum(x, axis=0, dtype=jnp.float32) * (1.0 / hw)
    h1 = jnp.dot(pooled, w1t_ref[...], preferred_element_type=jnp.float32)
    h1 = jnp.maximum(h1 + b1_ref[...], 0.0)
    h2 = jnp.dot(h1, w2t_ref[...], preferred_element_type=jnp.float32)
    gate = jax.nn.sigmoid(h2 + b2_ref[...])
    y_ref[...] = gate.astype(y_ref.dtype)[None, :, :] * x


@functools.partial(jax.jit, static_argnames=("bt",))
def _spatial_se(x, w1t, b1, w2t, b2, *, bt):
    B, C, H, W = x.shape
    hidden = w1t.shape[1]
    HW = H * W

    xt = jnp.transpose(x, (2, 3, 0, 1)).reshape(HW, B, C)
    nb = pl.cdiv(B, bt)

    yt = pl.pallas_call(
        _se_hwbc_kernel,
        out_shape=jax.ShapeDtypeStruct((HW, B, C), x.dtype),
        grid=(nb,),
        in_specs=[
            pl.BlockSpec((HW, bt, C), lambda b: (0, b, 0)),
            pl.BlockSpec((C, hidden), lambda b: (0, 0)),
            pl.BlockSpec((1, hidden), lambda b: (0, 0)),
            pl.BlockSpec((hidden, C), lambda b: (0, 0)),
            pl.BlockSpec((1, C), lambda b: (0, 0)),
        ],
        out_specs=pl.BlockSpec((HW, bt, C), lambda b: (0, b, 0)),
        compiler_params=pltpu.CompilerParams(
            dimension_semantics=("parallel",),
            vmem_limit_bytes=_VMEM_LIMIT_BYTES),
    )(xt, w1t, b1, w2t, b2)

    return yt.reshape(H, W, B, C).transpose(2, 3, 0, 1)


def kernel(x, w1t, b1, w2t, b2):
    return _spatial_se(x, w1t, b1, w2t, b2, bt=32)
